# baseline (speedup 1.0000x reference)
import jax
import jax.numpy as jnp
from jax import lax
from jax.experimental import pallas as pl
from jax.experimental.pallas import tpu as pltpu
from jax.experimental.pallas import tpu_sc as plsc

_NC = 2
_NS = 16
_NW = _NC * _NS
_CHUNK = 128


def _sc_agg_kernel(x_hbm, src_hbm, dst_hbm, zeros_hbm, out_hbm,
                   src_v, dst_v, buf0, buf1, acc, sem0, sem1):
    ch = src_hbm.shape[0] // _NW
    hch = src_v.shape[0]
    rpt = zeros_hbm.shape[0]
    c = lax.axis_index("c")
    s = lax.axis_index("s")
    wid = s * _NC + c

    pltpu.sync_copy(zeros_hbm, acc.at[pl.ds(s * rpt, rpt)])
    plsc.subcore_barrier()

    for h in range(ch // hch):
        off = wid * ch + h * hch
        pltpu.sync_copy(src_hbm.at[pl.ds(off, hch)], src_v)
        pltpu.sync_copy(dst_hbm.at[pl.ds(off, hch)], dst_v)

        pltpu.async_copy(x_hbm.at[src_v.at[0]], buf0, sem0)

        def body(i, _):
            j = i * 2
            pltpu.async_copy(x_hbm.at[src_v.at[j + 1]], buf1, sem1)
            pltpu.make_async_copy(x_hbm.at[src_v.at[j]], buf0, sem0).wait()
            pltpu.sync_copy(buf0, acc.at[dst_v.at[j]], add=True)

            @pl.when(j + 2 < hch)
            def _():
                pltpu.async_copy(x_hbm.at[src_v.at[j + 2]], buf0, sem0)

            pltpu.make_async_copy(x_hbm.at[src_v.at[j + 1]], buf1, sem1).wait()
            pltpu.sync_copy(buf1, acc.at[dst_v.at[j + 1]], add=True)
            return _

        lax.fori_loop(0, hch // 2, body, None)

    plsc.subcore_barrier()

    pltpu.sync_copy(acc.at[pl.ds(s * rpt, rpt)],
                    out_hbm.at[c, pl.ds(s * rpt, rpt)])


def _mlp_body(gate_ref, x_ref, agg_ref, w1_ref, b1_ref, w2_ref, b2_ref,
              y_ref):
    h = x_ref[...] + agg_ref[0] + agg_ref[1]
    hid = jnp.dot(h, w1_ref[...], preferred_element_type=jnp.float32)
    hid = jnp.maximum(hid + b1_ref[...], 0.0)
    y = jnp.dot(hid, w2_ref[...], preferred_element_type=jnp.float32)
    y_ref[...] = (y + b2_ref[...]) * gate_ref[0]


def kernel(x, edge_index, W1, b1, W2, b2, alpha):
    n, d = x.shape
    e = edge_index.shape[1]

    nch = e // _CHUNK
    ch = 16 * (-(-nch // (_NW * 16)))
    rows_pad = _NW * ch
    rpt = 8 * (-(-n // (_NS * 8)))
    n_pad = rpt * _NS
    while n_pad - n < 128:
        rpt += 8
        n_pad = rpt * _NS

    br = 512
    n_dummy = n_pad - n

    def _reformat_body(s_ref, d_ref, so_ref, do_ref):
        i = pl.program_id(0)
        row = i * br + jax.lax.broadcasted_iota(jnp.int32, (br, _CHUNK), 0)
        flat = row * _CHUNK + jax.lax.broadcasted_iota(
            jnp.int32, (br, _CHUNK), 1)
        real = row < nch
        so_ref[...] = jnp.where(real, s_ref[...], flat % n)
        do_ref[...] = jnp.where(real, d_ref[...], n + flat % n_dummy)

    src_pad, dst_pad = pl.pallas_call(
        _reformat_body,
        grid=(rows_pad // br,),
        in_specs=[pl.BlockSpec((br, _CHUNK), lambda i: (i, 0)),
                  pl.BlockSpec((br, _CHUNK), lambda i: (i, 0))],
        out_specs=[pl.BlockSpec((br, _CHUNK), lambda i: (i, 0)),
                   pl.BlockSpec((br, _CHUNK), lambda i: (i, 0))],
        out_shape=[jax.ShapeDtypeStruct((rows_pad, _CHUNK), jnp.int32),
                   jax.ShapeDtypeStruct((rows_pad, _CHUNK), jnp.int32)],
    )(edge_index[0].reshape(nch, _CHUNK), edge_index[1].reshape(nch, _CHUNK))

    zeros_init = jnp.zeros((rpt, d), jnp.float32)

    hch = ch // 2
    sc_agg = pl.kernel(
        _sc_agg_kernel,
        out_type=jax.ShapeDtypeStruct((_NC, n_pad, d), jnp.float32),
        mesh=plsc.VectorSubcoreMesh(core_axis_name="c", subcore_axis_name="s"),
        scratch_types=[
            pltpu.VMEM((hch, _CHUNK), jnp.int32),
            pltpu.VMEM((hch, _CHUNK), jnp.int32),
            pltpu.VMEM((_CHUNK, d), jnp.float32),
            pltpu.VMEM((_CHUNK, d), jnp.float32),
            pltpu.VMEM_SHARED((n_pad, d), jnp.float32),
            pltpu.SemaphoreType.DMA,
            pltpu.SemaphoreType.DMA,
        ],
    )
    agg2 = sc_agg(x, src_pad, dst_pad, zeros_init)

    gate = jax.nn.sigmoid(alpha)

    bn = 2000
    grid = -(-n // bn)
    y = pl.pallas_call(
        _mlp_body,
        grid=(grid,),
        in_specs=[
            pl.BlockSpec(memory_space=pltpu.SMEM),
            pl.BlockSpec((bn, d), lambda i: (i, 0)),
            pl.BlockSpec((_NC, bn, d), lambda i: (0, i, 0)),
            pl.BlockSpec((d, d), lambda i: (0, 0)),
            pl.BlockSpec((1, d), lambda i: (0, 0)),
            pl.BlockSpec((d, d), lambda i: (0, 0)),
            pl.BlockSpec((1, d), lambda i: (0, 0)),
        ],
        out_specs=pl.BlockSpec((bn, d), lambda i: (i, 0)),
        out_shape=jax.ShapeDtypeStruct((n, d), jnp.float32),
    )(gate, x, agg2, W1, b1.reshape(1, d), W2, b2.reshape(1, d))

    return (y, gate)


# speedup vs baseline: 1.0681x; 1.0681x over previous
import jax
import jax.numpy as jnp
from jax import lax
from jax.experimental import pallas as pl
from jax.experimental.pallas import tpu as pltpu
from jax.experimental.pallas import tpu_sc as plsc

_NC = 2
_NS = 16
_NW = _NC * _NS
_CHUNK = 128


def _sc_agg_kernel(x_hbm, ei_hbm, zeros_hbm, out_hbm,
                   src_v, dst_v, buf0, buf1, acc, sem0, sem1):
    ch = ei_hbm.shape[1] // _NW
    hch = src_v.shape[0]
    rpt = zeros_hbm.shape[0]
    c = lax.axis_index("c")
    s = lax.axis_index("s")
    wid = s * _NC + c

    pltpu.sync_copy(zeros_hbm, acc.at[pl.ds(s * rpt, rpt)])
    plsc.subcore_barrier()

    for h in range(ch // hch):
        off = wid * ch + h * hch
        pltpu.sync_copy(ei_hbm.at[0, pl.ds(off, hch)], src_v)
        pltpu.sync_copy(ei_hbm.at[1, pl.ds(off, hch)], dst_v)

        pltpu.async_copy(x_hbm.at[src_v.at[0]], buf0, sem0)

        def body(i, _):
            j = i * 2
            pltpu.async_copy(x_hbm.at[src_v.at[j + 1]], buf1, sem1)
            pltpu.make_async_copy(x_hbm.at[src_v.at[j]], buf0, sem0).wait()
            pltpu.sync_copy(buf0, acc.at[dst_v.at[j]], add=True)

            @pl.when(j + 2 < hch)
            def _():
                pltpu.async_copy(x_hbm.at[src_v.at[j + 2]], buf0, sem0)

            pltpu.make_async_copy(x_hbm.at[src_v.at[j + 1]], buf1, sem1).wait()
            pltpu.sync_copy(buf1, acc.at[dst_v.at[j + 1]], add=True)
            return _

        lax.fori_loop(0, hch // 2, body, None)

    plsc.subcore_barrier()

    pltpu.sync_copy(acc.at[pl.ds(s * rpt, rpt)],
                    out_hbm.at[c, pl.ds(s * rpt, rpt)])


def _mlp_body(gate_ref, x_ref, agg_ref, w1_ref, b1_ref, w2_ref, b2_ref,
              y_ref):
    h = x_ref[...] + agg_ref[0] + agg_ref[1]
    hid = jnp.dot(h, w1_ref[...], preferred_element_type=jnp.float32)
    hid = jnp.maximum(hid + b1_ref[...], 0.0)
    y = jnp.dot(hid, w2_ref[...], preferred_element_type=jnp.float32)
    y_ref[...] = (y + b2_ref[...]) * gate_ref[0]


def kernel(x, edge_index, W1, b1, W2, b2, alpha):
    n, d = x.shape
    e = edge_index.shape[1]

    nch = e // _CHUNK
    ch = 16 * (-(-nch // (_NW * 16)))
    rows_pad = _NW * ch
    rpt = 8 * (-(-n // (_NS * 8)))
    n_pad = rpt * _NS
    while n_pad - n < 128:
        rpt += 8
        n_pad = rpt * _NS

    br = 512
    n_dummy = n_pad - n

    def _reformat_body(ei_ref, out_ref):
        i = pl.program_id(0)
        row = i * br + jax.lax.broadcasted_iota(jnp.int32, (br, _CHUNK), 0)
        flat = row * _CHUNK + jax.lax.broadcasted_iota(
            jnp.int32, (br, _CHUNK), 1)
        real = row < nch
        out_ref[0] = jnp.where(real, ei_ref[0], flat % n)
        out_ref[1] = jnp.where(real, ei_ref[1], n + flat % n_dummy)

    ei = pl.pallas_call(
        _reformat_body,
        grid=(rows_pad // br,),
        in_specs=[pl.BlockSpec((2, br, _CHUNK), lambda i: (0, i, 0))],
        out_specs=pl.BlockSpec((2, br, _CHUNK), lambda i: (0, i, 0)),
        out_shape=jax.ShapeDtypeStruct((2, rows_pad, _CHUNK), jnp.int32),
    )(edge_index.reshape(2, nch, _CHUNK))

    zeros_init = jnp.zeros((rpt, d), jnp.float32)

    hch = ch // 2
    sc_agg = pl.kernel(
        _sc_agg_kernel,
        out_type=jax.ShapeDtypeStruct((_NC, n_pad, d), jnp.float32),
        mesh=plsc.VectorSubcoreMesh(core_axis_name="c", subcore_axis_name="s"),
        scratch_types=[
            pltpu.VMEM((hch, _CHUNK), jnp.int32),
            pltpu.VMEM((hch, _CHUNK), jnp.int32),
            pltpu.VMEM((_CHUNK, d), jnp.float32),
            pltpu.VMEM((_CHUNK, d), jnp.float32),
            pltpu.VMEM_SHARED((n_pad, d), jnp.float32),
            pltpu.SemaphoreType.DMA,
            pltpu.SemaphoreType.DMA,
        ],
    )
    agg2 = sc_agg(x, ei, zeros_init)

    gate = jax.nn.sigmoid(alpha)

    bn = 2000
    grid = -(-n // bn)
    y = pl.pallas_call(
        _mlp_body,
        grid=(grid,),
        in_specs=[
            pl.BlockSpec(memory_space=pltpu.SMEM),
            pl.BlockSpec((bn, d), lambda i: (i, 0)),
            pl.BlockSpec((_NC, bn, d), lambda i: (0, i, 0)),
            pl.BlockSpec((d, d), lambda i: (0, 0)),
            pl.BlockSpec((1, d), lambda i: (0, 0)),
            pl.BlockSpec((d, d), lambda i: (0, 0)),
            pl.BlockSpec((1, d), lambda i: (0, 0)),
        ],
        out_specs=pl.BlockSpec((bn, d), lambda i: (i, 0)),
        out_shape=jax.ShapeDtypeStruct((n, d), jnp.float32),
    )(gate, x, agg2, W1, b1.reshape(1, d), W2, b2.reshape(1, d))

    return (y, gate)


# speedup vs baseline: 1.0706x; 1.0023x over previous
import functools

import jax
import jax.numpy as jnp
from jax import lax
from jax.experimental import pallas as pl
from jax.experimental.pallas import tpu as pltpu
from jax.experimental.pallas import tpu_sc as plsc

_NC = 2
_NS = 16
_NW = _NC * _NS
_CHUNK = 128


def _sc_agg_kernel(nch, x_hbm, ei_hbm, zeros_hbm, out_hbm,
                    src_v, dst_v, buf0, buf1, acc, sem0, sem1):
    ch = ei_hbm.shape[1] // _NW
    hch = src_v.shape[0]
    rpt = zeros_hbm.shape[0]
    c = lax.axis_index("c")
    s = lax.axis_index("s")
    wid = s * _NC + c

    pltpu.sync_copy(zeros_hbm, acc.at[pl.ds(s * rpt, rpt)])
    plsc.subcore_barrier()

    cnt = jnp.clip(nch - wid * ch, 0, ch)

    for h in range(ch // hch):
        off = wid * ch + h * hch
        m = jnp.clip(cnt - h * hch, 0, hch)

        @pl.when(m > 0)
        def _():
            pltpu.sync_copy(ei_hbm.at[0, pl.ds(off, hch)], src_v)
            pltpu.sync_copy(ei_hbm.at[1, pl.ds(off, hch)], dst_v)

            pltpu.async_copy(x_hbm.at[src_v.at[0]], buf0, sem0)

            def body(i, _):
                j = i * 2

                @pl.when(j + 1 < m)
                def _():
                    pltpu.async_copy(x_hbm.at[src_v.at[j + 1]], buf1, sem1)

                pltpu.make_async_copy(x_hbm.at[src_v.at[j]], buf0,
                                      sem0).wait()
                pltpu.sync_copy(buf0, acc.at[dst_v.at[j]], add=True)

                @pl.when(j + 2 < m)
                def _():
                    pltpu.async_copy(x_hbm.at[src_v.at[j + 2]], buf0, sem0)

                @pl.when(j + 1 < m)
                def _():
                    pltpu.make_async_copy(x_hbm.at[src_v.at[j + 1]], buf1,
                                          sem1).wait()
                    pltpu.sync_copy(buf1, acc.at[dst_v.at[j + 1]], add=True)
                return _

            lax.fori_loop(0, (m + 1) // 2, body, None)

    plsc.subcore_barrier()

    pltpu.sync_copy(acc.at[pl.ds(s * rpt, rpt)],
                    out_hbm.at[c, pl.ds(s * rpt, rpt)])


def _mlp_body(gate_ref, x_ref, agg_ref, w1_ref, b1_ref, w2_ref, b2_ref,
              y_ref):
    h = x_ref[...] + agg_ref[0] + agg_ref[1]
    hid = jnp.dot(h, w1_ref[...], preferred_element_type=jnp.float32)
    hid = jnp.maximum(hid + b1_ref[...], 0.0)
    y = jnp.dot(hid, w2_ref[...], preferred_element_type=jnp.float32)
    y_ref[...] = (y + b2_ref[...]) * gate_ref[0]


def kernel(x, edge_index, W1, b1, W2, b2, alpha):
    n, d = x.shape
    e = edge_index.shape[1]

    nch = e // _CHUNK
    ch = 16 * (-(-nch // (_NW * 16)))
    rows_pad = _NW * ch
    rpt = 8 * (-(-n // (_NS * 8)))
    n_pad = rpt * _NS
    while n_pad - n < 128:
        rpt += 8
        n_pad = rpt * _NS

    br = 512
    n_dummy = n_pad - n

    def _reformat_body(ei_ref, out_ref):
        i = pl.program_id(0)
        row = i * br + jax.lax.broadcasted_iota(jnp.int32, (br, _CHUNK), 0)
        flat = row * _CHUNK + jax.lax.broadcasted_iota(
            jnp.int32, (br, _CHUNK), 1)
        real = row < nch
        out_ref[0] = jnp.where(real, ei_ref[0], flat % n)
        out_ref[1] = jnp.where(real, ei_ref[1], n + flat % n_dummy)

    ei = pl.pallas_call(
        _reformat_body,
        grid=(rows_pad // br,),
        in_specs=[pl.BlockSpec((2, br, _CHUNK), lambda i: (0, i, 0))],
        out_specs=pl.BlockSpec((2, br, _CHUNK), lambda i: (0, i, 0)),
        out_shape=jax.ShapeDtypeStruct((2, rows_pad, _CHUNK), jnp.int32),
    )(edge_index.reshape(2, nch, _CHUNK))

    zeros_init = jnp.zeros((rpt, d), jnp.float32)

    hch = ch // 2
    sc_agg = pl.kernel(
        functools.partial(_sc_agg_kernel, nch),
        out_type=jax.ShapeDtypeStruct((_NC, n_pad, d), jnp.float32),
        mesh=plsc.VectorSubcoreMesh(core_axis_name="c", subcore_axis_name="s"),
        scratch_types=[
            pltpu.VMEM((hch, _CHUNK), jnp.int32),
            pltpu.VMEM((hch, _CHUNK), jnp.int32),
            pltpu.VMEM((_CHUNK, d), jnp.float32),
            pltpu.VMEM((_CHUNK, d), jnp.float32),
            pltpu.VMEM_SHARED((n_pad, d), jnp.float32),
            pltpu.SemaphoreType.DMA,
            pltpu.SemaphoreType.DMA,
        ],
    )
    agg2 = sc_agg(x, ei, zeros_init)

    gate = jax.nn.sigmoid(alpha)

    bn = 2000
    grid = -(-n // bn)
    y = pl.pallas_call(
        _mlp_body,
        grid=(grid,),
        in_specs=[
            pl.BlockSpec(memory_space=pltpu.SMEM),
            pl.BlockSpec((bn, d), lambda i: (i, 0)),
            pl.BlockSpec((_NC, bn, d), lambda i: (0, i, 0)),
            pl.BlockSpec((d, d), lambda i: (0, 0)),
            pl.BlockSpec((1, d), lambda i: (0, 0)),
            pl.BlockSpec((d, d), lambda i: (0, 0)),
            pl.BlockSpec((1, d), lambda i: (0, 0)),
        ],
        out_specs=pl.BlockSpec((bn, d), lambda i: (i, 0)),
        out_shape=jax.ShapeDtypeStruct((n, d), jnp.float32),
    )(gate, x, agg2, W1, b1.reshape(1, d), W2, b2.reshape(1, d))

    return (y, gate)
